# baseline (speedup 1.0000x reference)
import jax
import jax.numpy as jnp
from jax import lax
from jax.experimental import pallas as pl
from jax.experimental.pallas import tpu as pltpu
from jax.experimental.pallas import tpu_sc as plsc

NUM_EMBEDS = 1000000
DIM = 64
NUM_CONCEPTS = 4
LANES = 16
NUM_CORES = 2
NUM_SUBCORES = 16
NUM_WORKERS = NUM_CORES * NUM_SUBCORES

CHUNK = 512
GATHER = 128
QDIM = DIM // 2
GATHERS_PER_CHUNK = 2 * CHUNK // GATHER
NBUF = 2


def _body(x_hbm, emb_hbm, conc_hbm, out_hbm,
          idx_raw, idx_flt, cidb, hitf, rows, conc_v,
          sem_idx0, sem_idx1, sem_g0, sem_g1, sem_s0, sem_s1):
    sem_idx = (sem_idx0, sem_idx1)
    sem_g = (sem_g0, sem_g1)
    sem_s = (sem_s0, sem_s1)
    n = x_hbm.shape[0]
    per_worker = n // NUM_WORKERS
    chunks = per_worker // CHUNK

    wid = lax.axis_index("s") * NUM_CORES + lax.axis_index("c")
    base0 = wid * per_worker

    pltpu.sync_copy(conc_hbm, conc_v)

    def start_idx(g, b):
        pltpu.async_copy(x_hbm.at[pl.ds(base0 + g * CHUNK, CHUNK)],
                         idx_raw.at[b], sem_idx[b])

    def drain_idx(b):
        pltpu.make_async_copy(x_hbm.at[pl.ds(0, CHUNK)], idx_raw.at[b],
                              sem_idx[b]).wait()

    def prep(g, b):
        drain_idx(b)
        acc = jnp.zeros((LANES,), jnp.bool_)
        lanes = lax.iota(jnp.int32, LANES)
        for i in range(CHUNK // LANES):
            v = idx_raw[b, pl.ds(i * LANES, LANES)]
            is_c = v >= NUM_EMBEDS
            vf = jnp.where(is_c, 0, v)
            pos2 = (i * LANES + lanes) * 2
            plsc.store_scatter(idx_flt.at[b], [pos2], vf * 4)
            plsc.store_scatter(idx_flt.at[b], [pos2 + 1], vf * 4 + 1)
            cidb[b, pl.ds(i * LANES, LANES)] = jnp.where(is_c, v - NUM_EMBEDS, -1)
            acc = acc | is_c
        hitf[b, pl.ds(0, LANES)] = jnp.where(acc, 1, 0)
        for j in range(GATHERS_PER_CHUNK):
            pltpu.async_copy(
                emb_hbm.at[idx_flt.at[b, pl.ds(j * GATHER, GATHER)]],
                rows.at[b, pl.ds(j * GATHER, GATHER)],
                sem_g[b])

    def finish(g, b):
        for j in range(GATHERS_PER_CHUNK):
            pltpu.make_async_copy(
                emb_hbm.at[idx_flt.at[b, pl.ds(j * GATHER, GATHER)]],
                rows.at[b, pl.ds(j * GATHER, GATHER)],
                sem_g[b]).wait()

        accv = hitf[b, pl.ds(0, LANES)]
        hits = accv[0]
        for r in range(1, LANES):
            hits = hits | accv[r]

        @pl.when(hits > 0)
        def _fixup():
            def fix_group(i, _):
                lanepos = lax.iota(jnp.int32, LANES) + i * LANES
                vc = cidb[b, pl.ds(i * LANES, LANES)]
                mask = vc >= 0
                cid = jnp.maximum(vc, 0)
                for c in range(DIM):
                    col = jnp.full((LANES,), c, jnp.int32)
                    vals = plsc.load_gather(conc_v, [cid, col])
                    plsc.store_scatter(
                        rows.at[b],
                        [lanepos * 2 + (c // QDIM), col - (c // QDIM) * QDIM],
                        vals, mask=mask)
                return 0

            lax.fori_loop(0, CHUNK // LANES, fix_group, 0)

        pltpu.async_copy(rows.at[b],
                         out_hbm.at[pl.ds(2 * (base0 + g * CHUNK), 2 * CHUNK)],
                         sem_s[b])

    def drain_scatter(b):
        pltpu.make_async_copy(out_hbm.at[pl.ds(0, 2 * CHUNK)], rows.at[b],
                              sem_s[b]).wait()

    start_idx(0, 0)
    start_idx(1, 1)
    prep(0, 0)
    start_idx(2, 0)
    prep(1, 1)
    start_idx(3, 1)

    def pair_body(i, _):
        g0 = 2 * i
        finish(g0, 0)
        finish(g0 + 1, 1)
        drain_scatter(0)
        prep(g0 + 2, 0)
        start_idx(g0 + 4, 0)
        drain_scatter(1)
        prep(g0 + 3, 1)
        start_idx(g0 + 5, 1)
        return 0

    lax.fori_loop(0, chunks // 2 - 1, pair_body, 0)

    finish(chunks - 2, 0)
    finish(chunks - 1, 1)
    drain_idx(0)
    drain_idx(1)
    drain_scatter(0)
    drain_scatter(1)


def _tc_transform(lin128, b, s):
    bb = b // 128

    def body(in_ref, i64_ref, o_ref):
        a = in_ref[...]
        ident = i64_ref[...]
        ev = a[:, :DIM]
        od = a[:, DIM:]
        evt = lax.dot_general(ev, ident, (((0,), (0,)), ((), ())),
                              preferred_element_type=jnp.float32)
        odt = lax.dot_general(od, ident, (((0,), (0,)), ((), ())),
                              preferred_element_type=jnp.float32)
        out = jnp.stack([evt, odt], axis=2).reshape(DIM, 128)
        o_ref[0] = out

    ident = jnp.eye(DIM, dtype=jnp.float32)
    return pl.pallas_call(
        body,
        grid=(s, bb),
        in_specs=[
            pl.BlockSpec((64, 128), lambda si, bi: (si * bb + bi, 0)),
            pl.BlockSpec((DIM, DIM), lambda si, bi: (0, 0)),
        ],
        out_specs=pl.BlockSpec((1, DIM, 128), lambda si, bi: (si, 0, bi)),
        out_shape=jax.ShapeDtypeStruct((s, DIM, b), jnp.float32),
    )(lin128, ident)


def kernel(x, embed_weight, concepts):
    b, s = x.shape
    n = b * s
    x_flat = x.T.reshape(n)
    from jax.experimental.layout import Layout, with_layout_constraint

    embp = jnp.pad(embed_weight, ((0, 0), (0, DIM)))
    embp = with_layout_constraint(embp, Layout((0, 1), tiling=((8, 128),)))
    emb4 = embp.reshape(4 * NUM_EMBEDS, QDIM)

    mesh = plsc.VectorSubcoreMesh(core_axis_name="c", subcore_axis_name="s",
                                  num_cores=NUM_CORES, num_subcores=NUM_SUBCORES)
    out = pl.kernel(
        _body,
        out_type=jax.ShapeDtypeStruct((2 * n, QDIM), jnp.float32),
        mesh=mesh,
        scratch_types=[
            pltpu.VMEM((NBUF, CHUNK), jnp.int32),
            pltpu.VMEM((NBUF, 2 * CHUNK), jnp.int32),
            pltpu.VMEM((NBUF, CHUNK), jnp.int32),
            pltpu.VMEM((NBUF, LANES), jnp.int32),
            pltpu.VMEM((NBUF, 2 * CHUNK, QDIM), jnp.float32),
            pltpu.VMEM((NUM_CONCEPTS, DIM), jnp.float32),
            pltpu.SemaphoreType.DMA,
            pltpu.SemaphoreType.DMA,
            pltpu.SemaphoreType.DMA,
            pltpu.SemaphoreType.DMA,
            pltpu.SemaphoreType.DMA,
            pltpu.SemaphoreType.DMA,
        ],
        compiler_params=pltpu.CompilerParams(use_tc_tiling_on_sc=False,
                                             needs_layout_passes=False),
    )(x_flat, emb4, concepts)
    lin128 = out.reshape(n // 2, 2 * DIM)
    native = _tc_transform(lin128, b, s)
    native = with_layout_constraint(
        native, Layout((0, 1, 2), tiling=((8, 128),)))
    res = native.transpose((2, 0, 1))
    res = with_layout_constraint(res, Layout((1, 2, 0), tiling=((8, 128),)))
    return res


# speedup vs baseline: 1.1157x; 1.1157x over previous
import jax
import jax.numpy as jnp
from jax import lax
from jax.experimental import pallas as pl
from jax.experimental.pallas import tpu as pltpu
from jax.experimental.pallas import tpu_sc as plsc

NUM_EMBEDS = 1000000
DIM = 64
NUM_CONCEPTS = 4
LANES = 16
NUM_CORES = 2
NUM_SUBCORES = 16
NUM_WORKERS = NUM_CORES * NUM_SUBCORES

CHUNK = 512
GATHER = 128
QDIM = DIM // 2
GATHERS_PER_CHUNK = 2 * CHUNK // GATHER
NBUF = 2


def _body(x_hbm, emb_hbm, conc_hbm, out_hbm,
          idx_raw, idx_flt, cidb, hitf, rows, conc_v,
          sem_idx0, sem_idx1, sem_g0, sem_g1, sem_s0, sem_s1):
    sem_idx = (sem_idx0, sem_idx1)
    sem_g = (sem_g0, sem_g1)
    sem_s = (sem_s0, sem_s1)
    n = x_hbm.shape[0]
    per_worker = n // NUM_WORKERS
    chunks = per_worker // CHUNK

    wid = lax.axis_index("s") * NUM_CORES + lax.axis_index("c")
    base0 = wid * per_worker

    pltpu.sync_copy(conc_hbm, conc_v)

    def start_idx(g, b):
        pltpu.async_copy(x_hbm.at[pl.ds(base0 + g * CHUNK, CHUNK)],
                         idx_raw.at[b], sem_idx[b])

    def drain_idx(b):
        pltpu.make_async_copy(x_hbm.at[pl.ds(0, CHUNK)], idx_raw.at[b],
                              sem_idx[b]).wait()

    def prep(g, b):
        drain_idx(b)
        acc = jnp.zeros((LANES,), jnp.bool_)
        lanes = lax.iota(jnp.int32, LANES)
        for i in range(CHUNK // LANES):
            v = idx_raw[b, pl.ds(i * LANES, LANES)]
            is_c = v >= NUM_EMBEDS
            vf = jnp.where(is_c, 0, v)
            pos2 = (i * LANES + lanes) * 2
            plsc.store_scatter(idx_flt.at[b], [pos2], vf * 4)
            plsc.store_scatter(idx_flt.at[b], [pos2 + 1], vf * 4 + 1)
            cidb[b, pl.ds(i * LANES, LANES)] = jnp.where(is_c, v - NUM_EMBEDS, -1)
            acc = acc | is_c
        hitf[b, pl.ds(0, LANES)] = jnp.where(acc, 1, 0)
        for j in range(GATHERS_PER_CHUNK):
            pltpu.async_copy(
                emb_hbm.at[idx_flt.at[b, pl.ds(j * GATHER, GATHER)]],
                rows.at[b, pl.ds(j * GATHER, GATHER)],
                sem_g[b])

    def finish(g, b):
        for j in range(GATHERS_PER_CHUNK):
            pltpu.make_async_copy(
                emb_hbm.at[idx_flt.at[b, pl.ds(j * GATHER, GATHER)]],
                rows.at[b, pl.ds(j * GATHER, GATHER)],
                sem_g[b]).wait()

        accv = hitf[b, pl.ds(0, LANES)]
        hits = accv[0]
        for r in range(1, LANES):
            hits = hits | accv[r]

        @pl.when(hits > 0)
        def _fixup():
            def fix_group(i, _):
                lanepos = lax.iota(jnp.int32, LANES) + i * LANES
                vc = cidb[b, pl.ds(i * LANES, LANES)]
                mask = vc >= 0
                cid = jnp.maximum(vc, 0)
                for c in range(DIM):
                    col = jnp.full((LANES,), c, jnp.int32)
                    vals = plsc.load_gather(conc_v, [cid, col])
                    plsc.store_scatter(
                        rows.at[b],
                        [lanepos * 2 + (c // QDIM), col - (c // QDIM) * QDIM],
                        vals, mask=mask)
                return 0

            lax.fori_loop(0, CHUNK // LANES, fix_group, 0)

        pltpu.async_copy(rows.at[b],
                         out_hbm.at[pl.ds(2 * (base0 + g * CHUNK), 2 * CHUNK)],
                         sem_s[b])

    def drain_scatter(b):
        pltpu.make_async_copy(out_hbm.at[pl.ds(0, 2 * CHUNK)], rows.at[b],
                              sem_s[b]).wait()

    start_idx(0, 0)
    start_idx(1, 1)
    prep(0, 0)
    start_idx(2, 0)
    prep(1, 1)
    start_idx(3, 1)

    def pair_body(i, _):
        g0 = 2 * i
        finish(g0, 0)
        finish(g0 + 1, 1)
        drain_scatter(0)
        prep(g0 + 2, 0)
        start_idx(g0 + 4, 0)
        drain_scatter(1)
        prep(g0 + 3, 1)
        start_idx(g0 + 5, 1)
        return 0

    lax.fori_loop(0, chunks // 2 - 1, pair_body, 0)

    finish(chunks - 2, 0)
    finish(chunks - 1, 1)
    drain_idx(0)
    drain_idx(1)
    drain_scatter(0)
    drain_scatter(1)


def _tc_transform(lin128, b, s):
    rows_blk = 1024
    bb = b // (2 * rows_blk)

    def body(in_ref, o_ref):
        a = in_ref[...]
        ev = a[:, :DIM]
        od = a[:, DIM:]
        evt = ev.T
        odt = od.T
        o_ref[0] = jnp.stack([evt, odt], axis=2).reshape(DIM, 2 * rows_blk)

    return pl.pallas_call(
        body,
        grid=(s, bb),
        in_specs=[
            pl.BlockSpec((rows_blk, 128), lambda si, bi: (si * bb + bi, 0)),
        ],
        out_specs=pl.BlockSpec((1, DIM, 2 * rows_blk),
                               lambda si, bi: (si, 0, bi)),
        out_shape=jax.ShapeDtypeStruct((s, DIM, b), jnp.float32),
    )(lin128)


def kernel(x, embed_weight, concepts):
    b, s = x.shape
    n = b * s
    x_flat = x.T.reshape(n)
    from jax.experimental.layout import Layout, with_layout_constraint

    embp = jnp.pad(embed_weight, ((0, 0), (0, DIM)))
    embp = with_layout_constraint(embp, Layout((0, 1), tiling=((8, 128),)))
    emb4 = embp.reshape(4 * NUM_EMBEDS, QDIM)

    mesh = plsc.VectorSubcoreMesh(core_axis_name="c", subcore_axis_name="s",
                                  num_cores=NUM_CORES, num_subcores=NUM_SUBCORES)
    out = pl.kernel(
        _body,
        out_type=jax.ShapeDtypeStruct((2 * n, QDIM), jnp.float32),
        mesh=mesh,
        scratch_types=[
            pltpu.VMEM((NBUF, CHUNK), jnp.int32),
            pltpu.VMEM((NBUF, 2 * CHUNK), jnp.int32),
            pltpu.VMEM((NBUF, CHUNK), jnp.int32),
            pltpu.VMEM((NBUF, LANES), jnp.int32),
            pltpu.VMEM((NBUF, 2 * CHUNK, QDIM), jnp.float32),
            pltpu.VMEM((NUM_CONCEPTS, DIM), jnp.float32),
            pltpu.SemaphoreType.DMA,
            pltpu.SemaphoreType.DMA,
            pltpu.SemaphoreType.DMA,
            pltpu.SemaphoreType.DMA,
            pltpu.SemaphoreType.DMA,
            pltpu.SemaphoreType.DMA,
        ],
        compiler_params=pltpu.CompilerParams(use_tc_tiling_on_sc=False,
                                             needs_layout_passes=False),
    )(x_flat, emb4, concepts)
    lin128 = out.reshape(n // 2, 2 * DIM)
    native = _tc_transform(lin128, b, s)
    native = with_layout_constraint(
        native, Layout((0, 1, 2), tiling=((8, 128),)))
    res = native.transpose((2, 0, 1))
    res = with_layout_constraint(res, Layout((1, 2, 0), tiling=((8, 128),)))
    return res


# speedup vs baseline: 8.6665x; 7.7678x over previous
import jax
import jax.numpy as jnp
from jax import lax
from jax.experimental import pallas as pl
from jax.experimental.pallas import tpu as pltpu
from jax.experimental.pallas import tpu_sc as plsc

NUM_EMBEDS = 1000000
DIM = 64
NUM_CONCEPTS = 4
LANES = 16
NUM_CORES = 2
NUM_SUBCORES = 16
NUM_WORKERS = NUM_CORES * NUM_SUBCORES

CHUNK = 512
GATHER = 128
QDIM = DIM // 2
GATHERS_PER_CHUNK = 2 * CHUNK // GATHER
NBUF = 2


def _body(x_hbm, emb_hbm, conc_hbm, out_hbm,
          idx_raw, idx_flt, cidb, hitf, rows, conc_v,
          sem_idx0, sem_idx1, sem_g0, sem_g1, sem_s0, sem_s1):
    sem_idx = (sem_idx0, sem_idx1)
    sem_g = (sem_g0, sem_g1)
    sem_s = (sem_s0, sem_s1)
    n = x_hbm.shape[0]
    per_worker = n // NUM_WORKERS
    chunks = per_worker // CHUNK

    wid = lax.axis_index("s") * NUM_CORES + lax.axis_index("c")
    base0 = wid * per_worker

    pltpu.sync_copy(conc_hbm, conc_v)

    def start_idx(g, b):
        pltpu.async_copy(x_hbm.at[pl.ds(base0 + g * CHUNK, CHUNK)],
                         idx_raw.at[b], sem_idx[b])

    def drain_idx(b):
        pltpu.make_async_copy(x_hbm.at[pl.ds(0, CHUNK)], idx_raw.at[b],
                              sem_idx[b]).wait()

    def prep(g, b):
        drain_idx(b)
        acc = jnp.zeros((LANES,), jnp.bool_)
        lanes = lax.iota(jnp.int32, LANES)
        for i in range(CHUNK // LANES):
            v = idx_raw[b, pl.ds(i * LANES, LANES)]
            is_c = v >= NUM_EMBEDS
            vf = jnp.where(is_c, 0, v)
            pos2 = (i * LANES + lanes) * 2
            plsc.store_scatter(idx_flt.at[b], [pos2], vf * 4)
            plsc.store_scatter(idx_flt.at[b], [pos2 + 1], vf * 4 + 1)
            cidb[b, pl.ds(i * LANES, LANES)] = jnp.where(is_c, v - NUM_EMBEDS, -1)
            acc = acc | is_c
        hitf[b, pl.ds(0, LANES)] = jnp.where(acc, 1, 0)
        for j in range(GATHERS_PER_CHUNK):
            pltpu.async_copy(
                emb_hbm.at[idx_flt.at[b, pl.ds(j * GATHER, GATHER)]],
                rows.at[b, pl.ds(j * GATHER, GATHER)],
                sem_g[b])

    def finish(g, b):
        for j in range(GATHERS_PER_CHUNK):
            pltpu.make_async_copy(
                emb_hbm.at[idx_flt.at[b, pl.ds(j * GATHER, GATHER)]],
                rows.at[b, pl.ds(j * GATHER, GATHER)],
                sem_g[b]).wait()

        accv = hitf[b, pl.ds(0, LANES)]
        hits = accv[0]
        for r in range(1, LANES):
            hits = hits | accv[r]

        @pl.when(hits > 0)
        def _fixup():
            def fix_group(i, _):
                lanepos = lax.iota(jnp.int32, LANES) + i * LANES
                vc = cidb[b, pl.ds(i * LANES, LANES)]
                mask = vc >= 0
                cid = jnp.maximum(vc, 0)
                for c in range(DIM):
                    col = jnp.full((LANES,), c, jnp.int32)
                    vals = plsc.load_gather(conc_v, [cid, col])
                    plsc.store_scatter(
                        rows.at[b],
                        [lanepos * 2 + (c // QDIM), col - (c // QDIM) * QDIM],
                        vals, mask=mask)
                return 0

            lax.fori_loop(0, CHUNK // LANES, fix_group, 0)

        pltpu.async_copy(rows.at[b],
                         out_hbm.at[pl.ds(2 * (base0 + g * CHUNK), 2 * CHUNK)],
                         sem_s[b])

    def drain_scatter(b):
        pltpu.make_async_copy(out_hbm.at[pl.ds(0, 2 * CHUNK)], rows.at[b],
                              sem_s[b]).wait()

    start_idx(0, 0)
    start_idx(1, 1)
    prep(0, 0)
    start_idx(2, 0)
    prep(1, 1)
    start_idx(3, 1)

    def pair_body(i, _):
        g0 = 2 * i
        finish(g0, 0)
        finish(g0 + 1, 1)
        drain_scatter(0)
        prep(g0 + 2, 0)
        start_idx(g0 + 4, 0)
        drain_scatter(1)
        prep(g0 + 3, 1)
        start_idx(g0 + 5, 1)
        return 0

    lax.fori_loop(0, chunks // 2 - 1, pair_body, 0)

    finish(chunks - 2, 0)
    finish(chunks - 1, 1)
    drain_idx(0)
    drain_idx(1)
    drain_scatter(0)
    drain_scatter(1)


def _tc_transform(lin128, b, s):
    rows_blk = 1024
    half = b // 2
    bb = half // rows_blk

    def body(in_ref, o_ref):
        o_ref[0, :, 0] = in_ref[:, :DIM].T
        o_ref[0, :, 1] = in_ref[:, DIM:].T

    out4 = pl.pallas_call(
        body,
        grid=(s, bb),
        in_specs=[
            pl.BlockSpec((rows_blk, 128), lambda si, bi: (si * bb + bi, 0)),
        ],
        out_specs=pl.BlockSpec((1, DIM, 2, rows_blk),
                               lambda si, bi: (si, 0, 0, bi)),
        out_shape=jax.ShapeDtypeStruct((s, DIM, 2, half), jnp.float32),
    )(lin128)
    return out4.reshape(s, DIM, b)


def kernel(x, embed_weight, concepts):
    b, s = x.shape
    n = b * s
    x_flat = x.T.reshape(s, 2, b // 2).transpose(0, 2, 1).reshape(n)
    from jax.experimental.layout import Layout, with_layout_constraint

    embp = jnp.pad(embed_weight, ((0, 0), (0, DIM)))
    embp = with_layout_constraint(embp, Layout((0, 1), tiling=((8, 128),)))
    emb4 = embp.reshape(4 * NUM_EMBEDS, QDIM)

    mesh = plsc.VectorSubcoreMesh(core_axis_name="c", subcore_axis_name="s",
                                  num_cores=NUM_CORES, num_subcores=NUM_SUBCORES)
    out = pl.kernel(
        _body,
        out_type=jax.ShapeDtypeStruct((2 * n, QDIM), jnp.float32),
        mesh=mesh,
        scratch_types=[
            pltpu.VMEM((NBUF, CHUNK), jnp.int32),
            pltpu.VMEM((NBUF, 2 * CHUNK), jnp.int32),
            pltpu.VMEM((NBUF, CHUNK), jnp.int32),
            pltpu.VMEM((NBUF, LANES), jnp.int32),
            pltpu.VMEM((NBUF, 2 * CHUNK, QDIM), jnp.float32),
            pltpu.VMEM((NUM_CONCEPTS, DIM), jnp.float32),
            pltpu.SemaphoreType.DMA,
            pltpu.SemaphoreType.DMA,
            pltpu.SemaphoreType.DMA,
            pltpu.SemaphoreType.DMA,
            pltpu.SemaphoreType.DMA,
            pltpu.SemaphoreType.DMA,
        ],
        compiler_params=pltpu.CompilerParams(use_tc_tiling_on_sc=False,
                                             needs_layout_passes=False),
    )(x_flat, emb4, concepts)
    lin128 = out.reshape(n // 2, 2 * DIM)
    native = _tc_transform(lin128, b, s)
    native = with_layout_constraint(
        native, Layout((0, 1, 2), tiling=((8, 128),)))
    res = native.transpose((2, 0, 1))
    res = with_layout_constraint(res, Layout((1, 2, 0), tiling=((8, 128),)))
    return res


# speedup vs baseline: 10.8001x; 1.2462x over previous
import jax
import jax.numpy as jnp
from jax import lax
from jax.experimental import pallas as pl
from jax.experimental.pallas import tpu as pltpu
from jax.experimental.pallas import tpu_sc as plsc

NUM_EMBEDS = 1000000
DIM = 64
NUM_CONCEPTS = 4
LANES = 16
NUM_CORES = 2
NUM_SUBCORES = 16
NUM_WORKERS = NUM_CORES * NUM_SUBCORES

CHUNK = 512
GATHER = 128
QDIM = DIM // 2
GATHERS_PER_CHUNK = 2 * CHUNK // GATHER
NBUF = 2


def _body(x_hbm, emb_hbm, conc_hbm, out_hbm,
          idx_raw, idx_flt, cidb, hitf, rows, conc_v,
          sem_idx0, sem_idx1, sem_g0, sem_g1, sem_s0, sem_s1):
    sem_idx = (sem_idx0, sem_idx1)
    sem_g = (sem_g0, sem_g1)
    sem_s = (sem_s0, sem_s1)
    n = x_hbm.shape[0]
    per_worker = n // NUM_WORKERS
    chunks = per_worker // CHUNK

    wid = lax.axis_index("s") * NUM_CORES + lax.axis_index("c")
    base0 = wid * per_worker

    pltpu.sync_copy(conc_hbm, conc_v)

    def start_idx(g, b):
        pltpu.async_copy(x_hbm.at[pl.ds(base0 + g * CHUNK, CHUNK)],
                         idx_raw.at[b], sem_idx[b])

    def drain_idx(b):
        pltpu.make_async_copy(x_hbm.at[pl.ds(0, CHUNK)], idx_raw.at[b],
                              sem_idx[b]).wait()

    def prep(g, b):
        drain_idx(b)
        acc = jnp.zeros((LANES,), jnp.bool_)
        lanes = lax.iota(jnp.int32, LANES)
        for i in range(CHUNK // LANES):
            v = idx_raw[b, pl.ds(i * LANES, LANES)]
            is_c = v >= NUM_EMBEDS
            vf = jnp.where(is_c, 0, v)
            pos2 = (i * LANES + lanes) * 2
            plsc.store_scatter(idx_flt.at[b], [pos2], vf * 4)
            plsc.store_scatter(idx_flt.at[b], [pos2 + 1], vf * 4 + 1)
            cidb[b, pl.ds(i * LANES, LANES)] = jnp.where(is_c, v - NUM_EMBEDS, -1)
            acc = acc | is_c
        hitf[b, pl.ds(0, LANES)] = jnp.where(acc, 1, 0)
        for j in range(GATHERS_PER_CHUNK):
            pltpu.async_copy(
                emb_hbm.at[idx_flt.at[b, pl.ds(j * GATHER, GATHER)]],
                rows.at[b, pl.ds(j * GATHER, GATHER)],
                sem_g[b])

    def finish(g, b):
        for j in range(GATHERS_PER_CHUNK):
            pltpu.make_async_copy(
                emb_hbm.at[idx_flt.at[b, pl.ds(j * GATHER, GATHER)]],
                rows.at[b, pl.ds(j * GATHER, GATHER)],
                sem_g[b]).wait()

        accv = hitf[b, pl.ds(0, LANES)]
        hits = accv[0]
        for r in range(1, LANES):
            hits = hits | accv[r]

        @pl.when(hits > 0)
        def _fixup():
            def fix_group(i, _):
                lanepos = lax.iota(jnp.int32, LANES) + i * LANES
                vc = cidb[b, pl.ds(i * LANES, LANES)]
                mask = vc >= 0
                cid = jnp.maximum(vc, 0)
                for c in range(DIM):
                    col = jnp.full((LANES,), c, jnp.int32)
                    vals = plsc.load_gather(conc_v, [cid, col])
                    plsc.store_scatter(
                        rows.at[b],
                        [lanepos * 2 + (c // QDIM), col - (c // QDIM) * QDIM],
                        vals, mask=mask)
                return 0

            lax.fori_loop(0, CHUNK // LANES, fix_group, 0)

        pltpu.async_copy(rows.at[b],
                         out_hbm.at[pl.ds(2 * (base0 + g * CHUNK), 2 * CHUNK)],
                         sem_s[b])

    def drain_scatter(b):
        pltpu.make_async_copy(out_hbm.at[pl.ds(0, 2 * CHUNK)], rows.at[b],
                              sem_s[b]).wait()

    start_idx(0, 0)
    start_idx(1, 1)
    prep(0, 0)
    start_idx(2, 0)
    prep(1, 1)
    start_idx(3, 1)

    def pair_body(i, _):
        g0 = 2 * i
        finish(g0, 0)
        finish(g0 + 1, 1)
        drain_scatter(0)
        prep(g0 + 2, 0)
        start_idx(g0 + 4, 0)
        drain_scatter(1)
        prep(g0 + 3, 1)
        start_idx(g0 + 5, 1)
        return 0

    lax.fori_loop(0, chunks // 2 - 1, pair_body, 0)

    finish(chunks - 2, 0)
    finish(chunks - 1, 1)
    drain_idx(0)
    drain_idx(1)
    drain_scatter(0)
    drain_scatter(1)


def kernel(x, embed_weight, concepts):
    b, s = x.shape
    n = b * s
    x_flat = x.reshape(n)
    from jax.experimental.layout import Layout, with_layout_constraint

    embp = jnp.pad(embed_weight, ((0, 0), (0, DIM)))
    embp = with_layout_constraint(embp, Layout((0, 1), tiling=((8, 128),)))
    emb4 = embp.reshape(4 * NUM_EMBEDS, QDIM)

    mesh = plsc.VectorSubcoreMesh(core_axis_name="c", subcore_axis_name="s",
                                  num_cores=NUM_CORES, num_subcores=NUM_SUBCORES)
    out = pl.kernel(
        _body,
        out_type=jax.ShapeDtypeStruct((2 * n, QDIM), jnp.float32),
        mesh=mesh,
        scratch_types=[
            pltpu.VMEM((NBUF, CHUNK), jnp.int32),
            pltpu.VMEM((NBUF, 2 * CHUNK), jnp.int32),
            pltpu.VMEM((NBUF, CHUNK), jnp.int32),
            pltpu.VMEM((NBUF, LANES), jnp.int32),
            pltpu.VMEM((NBUF, 2 * CHUNK, QDIM), jnp.float32),
            pltpu.VMEM((NUM_CONCEPTS, DIM), jnp.float32),
            pltpu.SemaphoreType.DMA,
            pltpu.SemaphoreType.DMA,
            pltpu.SemaphoreType.DMA,
            pltpu.SemaphoreType.DMA,
            pltpu.SemaphoreType.DMA,
            pltpu.SemaphoreType.DMA,
        ],
        compiler_params=pltpu.CompilerParams(use_tc_tiling_on_sc=False,
                                             needs_layout_passes=False),
    )(x_flat, emb4, concepts)
    return out.reshape(b, s, DIM)
